# Initial kernel scaffold; baseline (speedup 1.0000x reference)
import jax
import jax.numpy as jnp
from jax.experimental import pallas as pl


def kernel(x, lut):
    raise NotImplementedError("write your pallas kernel here")


# speedup vs baseline: 1.2125x; 1.2125x over previous
import functools
import math

import jax
import jax.numpy as jnp
from jax import lax
from jax.experimental import pallas as pl
from jax.experimental.pallas import tpu as pltpu
from jax.experimental.pallas import tpu_sc as plsc

D_MODEL = 768
SCALE = math.sqrt(float(D_MODEL))

NUM_CORES = 2
NUM_SUBCORES = 16
NUM_WORKERS = NUM_CORES * NUM_SUBCORES
LANES = 16

CHUNK_ROWS = 64


@functools.partial(jax.jit, static_argnames=("b_total",))
def _embed_flat(x_flat, lut, *, b_total):
    d = lut.shape[1]
    b_per_w = b_total // NUM_WORKERS
    n_chunks = b_per_w // CHUNK_ROWS
    vecs_per_row = d // LANES

    mesh = plsc.VectorSubcoreMesh(
        core_axis_name="c", subcore_axis_name="s",
        num_cores=NUM_CORES, num_subcores=NUM_SUBCORES,
    )

    @functools.partial(
        pl.kernel,
        mesh=mesh,
        out_type=jax.ShapeDtypeStruct((b_total, d), jnp.float32),
        scratch_types=[
            pltpu.VMEM((b_per_w,), jnp.int32),
            pltpu.VMEM((CHUNK_ROWS, d), jnp.float32),
            pltpu.SemaphoreType.DMA,
        ],
    )
    def k(x_hbm, lut_hbm, out_hbm, idx_v, rows_v, sem):
        wid = lax.axis_index("s") * NUM_CORES + lax.axis_index("c")
        base = wid * b_per_w
        pltpu.sync_copy(x_hbm.at[pl.ds(base, b_per_w)], idx_v)

        def chunk_body(g, carry):
            pltpu.async_copy(
                lut_hbm.at[idx_v.at[pl.ds(g * CHUNK_ROWS, CHUNK_ROWS)]],
                rows_v, sem).wait()

            def row_body(r, carry2):
                for j in range(vecs_per_row):
                    sl = pl.ds(j * LANES, LANES)
                    rows_v[r, sl] = rows_v[r, sl] * SCALE
                return carry2

            lax.fori_loop(0, CHUNK_ROWS, row_body, 0, unroll=False)
            pltpu.sync_copy(
                rows_v, out_hbm.at[pl.ds(base + g * CHUNK_ROWS, CHUNK_ROWS)])
            return carry

        lax.fori_loop(0, n_chunks, chunk_body, 0, unroll=False)

    return k(x_flat, lut)


def kernel(x, lut):
    b_total = x.shape[0] * x.shape[1]
    out = _embed_flat(x.reshape(-1).astype(jnp.int32), lut, b_total=b_total)
    return out.reshape(x.shape + (lut.shape[1],))


# speedup vs baseline: 1.3495x; 1.1130x over previous
import functools
import math

import jax
import jax.numpy as jnp
from jax import lax
from jax.experimental import pallas as pl
from jax.experimental.pallas import tpu as pltpu
from jax.experimental.pallas import tpu_sc as plsc

D_MODEL = 768
SCALE = math.sqrt(float(D_MODEL))

NUM_CORES = 2
NUM_SUBCORES = 16
NUM_WORKERS = NUM_CORES * NUM_SUBCORES
LANES = 16

CHUNK_ROWS = 64


@functools.partial(jax.jit, static_argnames=("b_total",))
def _embed_flat(x_flat, lut, *, b_total):
    d = lut.shape[1]
    b_per_w = b_total // NUM_WORKERS
    n_chunks = b_per_w // CHUNK_ROWS
    n_steps = n_chunks // 2
    vecs_per_row = d // LANES

    mesh = plsc.VectorSubcoreMesh(
        core_axis_name="c", subcore_axis_name="s",
        num_cores=NUM_CORES, num_subcores=NUM_SUBCORES,
    )

    @functools.partial(
        pl.kernel,
        mesh=mesh,
        out_type=jax.ShapeDtypeStruct((b_total, d), jnp.float32),
        scratch_types=[
            pltpu.VMEM((b_per_w,), jnp.int32),
            pltpu.VMEM((CHUNK_ROWS, d), jnp.float32),
            pltpu.VMEM((CHUNK_ROWS, d), jnp.float32),
            pltpu.SemaphoreType.DMA,
            pltpu.SemaphoreType.DMA,
            pltpu.SemaphoreType.DMA,
            pltpu.SemaphoreType.DMA,
        ],
    )
    def k(x_hbm, lut_hbm, out_hbm, idx_v, rows0, rows1,
          gsem0, gsem1, osem0, osem1):
        wid = lax.axis_index("s") * NUM_CORES + lax.axis_index("c")
        base = wid * b_per_w
        pltpu.sync_copy(x_hbm.at[pl.ds(base, b_per_w)], idx_v)
        bufs = ((rows0, gsem0, osem0), (rows1, gsem1, osem1))

        def idx_slice(g):
            return idx_v.at[pl.ds(g * CHUNK_ROWS, CHUNK_ROWS)]

        def out_slice(g):
            return out_hbm.at[pl.ds(base + g * CHUNK_ROWS, CHUNK_ROWS)]

        def start_gather(g, buf, gsem):
            pltpu.async_copy(lut_hbm.at[idx_slice(g)], buf, gsem)

        def wait_gather(g, buf, gsem):
            pltpu.make_async_copy(lut_hbm.at[idx_slice(g)], buf, gsem).wait()

        def start_store(g, buf, osem):
            pltpu.async_copy(buf, out_slice(g), osem)

        def wait_store(g, buf, osem):
            pltpu.make_async_copy(buf, out_slice(g), osem).wait()

        def scale(buf):
            def row_body(r, carry):
                for j in range(vecs_per_row):
                    sl = pl.ds(j * LANES, LANES)
                    buf[r, sl] = buf[r, sl] * SCALE
                return carry
            lax.fori_loop(0, CHUNK_ROWS, row_body, 0, unroll=False)

        start_gather(0, rows0, gsem0)

        def step(s, carry):
            for b in range(2):
                g = 2 * s + b
                buf, gsem, osem = bufs[b]
                obuf, _, oosem = bufs[1 - b]
                wait_gather(g, buf, gsem)
                scale(buf)
                start_store(g, buf, osem)
                if b == 0:
                    @pl.when(s > 0)
                    def _():
                        wait_store(g - 1, obuf, oosem)
                    start_gather(g + 1, obuf, gsem1)
                else:
                    @pl.when(s < n_steps - 1)
                    def _():
                        wait_store(g - 1, obuf, oosem)
                        start_gather(g + 1, obuf, gsem0)
            return carry

        lax.fori_loop(0, n_steps, step, 0, unroll=False)
        wait_store(n_chunks - 2, rows0, osem0)
        wait_store(n_chunks - 1, rows1, osem1)

    return k(x_flat, lut)


def kernel(x, lut):
    b_total = x.shape[0] * x.shape[1]
    out = _embed_flat(x.reshape(-1).astype(jnp.int32), lut, b_total=b_total)
    return out.reshape(x.shape + (lut.shape[1],))


# speedup vs baseline: 1.6828x; 1.2470x over previous
import functools
import math

import jax
import jax.numpy as jnp
from jax import lax
from jax.experimental import pallas as pl
from jax.experimental.pallas import tpu as pltpu
from jax.experimental.pallas import tpu_sc as plsc

D_MODEL = 768
SCALE = math.sqrt(float(D_MODEL))

NUM_CORES = 2
NUM_SUBCORES = 16
NUM_WORKERS = NUM_CORES * NUM_SUBCORES
LANES = 16

CHUNK_ROWS = 64


@functools.partial(jax.jit, static_argnames=("b_total",))
def _embed_flat(x_flat, lut, *, b_total):
    d = lut.shape[1]
    b_per_w = b_total // NUM_WORKERS
    n_chunks = b_per_w // CHUNK_ROWS
    n_steps = n_chunks // 2
    vecs_per_row = d // LANES

    mesh = plsc.VectorSubcoreMesh(
        core_axis_name="c", subcore_axis_name="s",
        num_cores=NUM_CORES, num_subcores=NUM_SUBCORES,
    )

    @functools.partial(
        pl.kernel,
        mesh=mesh,
        out_type=jax.ShapeDtypeStruct((b_total, d), jnp.float32),
        scratch_types=[
            pltpu.VMEM((b_per_w,), jnp.int32),
            pltpu.VMEM((CHUNK_ROWS, d), jnp.float32),
            pltpu.VMEM((CHUNK_ROWS, d), jnp.float32),
            pltpu.SemaphoreType.DMA,
            pltpu.SemaphoreType.DMA,
            pltpu.SemaphoreType.DMA,
            pltpu.SemaphoreType.DMA,
        ],
    )
    def k(x_hbm, lut_hbm, out_hbm, idx_v, rows0, rows1,
          gsem0, gsem1, osem0, osem1):
        wid = lax.axis_index("s") * NUM_CORES + lax.axis_index("c")
        base = wid * b_per_w
        pltpu.sync_copy(x_hbm.at[pl.ds(base, b_per_w)], idx_v)
        bufs = ((rows0, gsem0, osem0), (rows1, gsem1, osem1))

        def idx_slice(g):
            return idx_v.at[pl.ds(g * CHUNK_ROWS, CHUNK_ROWS)]

        def out_slice(g):
            return out_hbm.at[pl.ds(base + g * CHUNK_ROWS, CHUNK_ROWS)]

        def start_gather(g, buf, gsem):
            pltpu.async_copy(lut_hbm.at[idx_slice(g)], buf, gsem)

        def wait_gather(g, buf, gsem):
            pltpu.make_async_copy(lut_hbm.at[idx_slice(g)], buf, gsem).wait()

        def start_store(g, buf, osem):
            pltpu.async_copy(buf, out_slice(g), osem)

        def wait_store(g, buf, osem):
            pltpu.make_async_copy(buf, out_slice(g), osem).wait()

        def scale(buf):
            def row_body(r, carry):
                for j in range(vecs_per_row):
                    sl = pl.ds(j * LANES, LANES)
                    buf[r, sl] = buf[r, sl] * SCALE
                return carry
            lax.fori_loop(0, CHUNK_ROWS, row_body, 0, unroll=False)

        start_gather(0, rows0, gsem0)

        def step(s, carry):
            for b in range(2):
                g = 2 * s + b
                buf, gsem, osem = bufs[b]
                obuf, _, oosem = bufs[1 - b]
                wait_gather(g, buf, gsem)
                if b == 0:
                    @pl.when(s > 0)
                    def _():
                        wait_store(g - 1, obuf, oosem)
                    start_gather(g + 1, obuf, gsem1)
                else:
                    @pl.when(s < n_steps - 1)
                    def _():
                        wait_store(g - 1, obuf, oosem)
                        start_gather(g + 1, obuf, gsem0)
                scale(buf)
                start_store(g, buf, osem)
            return carry

        lax.fori_loop(0, n_steps, step, 0, unroll=False)
        wait_store(n_chunks - 2, rows0, osem0)
        wait_store(n_chunks - 1, rows1, osem1)

    return k(x_flat, lut)


def kernel(x, lut):
    b_total = x.shape[0] * x.shape[1]
    out = _embed_flat(x.reshape(-1).astype(jnp.int32), lut, b_total=b_total)
    return out.reshape(x.shape + (lut.shape[1],))
